# baseline (speedup 1.0000x reference)
import functools

import jax
import jax.numpy as jnp
from jax import lax
from jax.experimental import pallas as pl
from jax.experimental.pallas import tpu as pltpu
from jax.experimental.pallas import tpu_sc as plsc


def kernel(input_ids, token_table, position_table):
    B, S = input_ids.shape
    V, D = token_table.shape
    N = B * S
    L = 16

    info = plsc.get_sparse_core_info()
    NC, NS = info.num_cores, info.num_subcores
    NW = NC * NS
    SB = S // NW
    NBUF = 4
    QB = SB // NBUF
    NCHUNK = NBUF * B
    AHEAD = 2

    ids_flat = input_ids.reshape(N).astype(jnp.int32)
    mesh = plsc.VectorSubcoreMesh(core_axis_name="c", subcore_axis_name="s")

    @functools.partial(
        pl.kernel,
        mesh=mesh,
        out_type=jax.ShapeDtypeStruct((N, D), jnp.float32),
        scratch_types=[
            pltpu.VMEM((B * SB,), jnp.int32),
            pltpu.VMEM((SB, D), jnp.float32),
        ]
        + [pltpu.VMEM((QB, D), jnp.float32) for _ in range(NBUF)]
        + [pltpu.SemaphoreType.DMA for _ in range(2 * NBUF + 2)],
    )
    def emb(ids_hbm, tok_hbm, pos_hbm, out_hbm, idx_v, pos_v, *rest):
        tok_bufs = rest[:NBUF]
        gsems = rest[NBUF : 2 * NBUF]
        ssems = rest[2 * NBUF : 3 * NBUF]
        isem = rest[3 * NBUF]
        psem = rest[3 * NBUF + 1]
        wid = lax.axis_index("s") * NC + lax.axis_index("c")
        s0 = wid * SB

        idx_h = [
            pltpu.async_copy(
                ids_hbm.at[pl.ds(b * S + s0, SB)],
                idx_v.at[pl.ds(b * SB, SB)],
                isem,
            )
            for b in range(B)
        ]
        pos_h = pltpu.async_copy(pos_hbm.at[pl.ds(s0, SB)], pos_v, psem)
        for h in idx_h:
            h.wait()

        def chunk_gather(i, buf):
            b, q = i // NBUF, i % NBUF
            return pltpu.async_copy(
                tok_hbm.at[idx_v.at[pl.ds(b * SB + q * QB, QB)]],
                tok_bufs[buf],
                gsems[buf],
            )

        gather_h = [None] * NBUF
        store_h = [None] * NBUF
        for i in range(AHEAD):
            gather_h[i] = chunk_gather(i, i)
        pos_h.wait()

        for i in range(NCHUNK):
            buf = i % NBUF
            if i + AHEAD < NCHUNK:
                ab = (i + AHEAD) % NBUF
                if store_h[ab] is not None:
                    store_h[ab].wait()
                    store_h[ab] = None
                gather_h[ab] = chunk_gather(i + AHEAD, ab)
            gather_h[buf].wait()

            b, q = i // NBUF, i % NBUF
            tok_v = tok_bufs[buf]

            NJ = D // L

            @plsc.parallel_loop(0, QB * NJ, unroll=4)
            def pair_add(t):
                r = t // NJ
                col = (t % NJ) * L
                plsc.addupdate(
                    tok_v.at[r, pl.ds(col, L)],
                    pos_v[q * QB + r, pl.ds(col, L)],
                )

            store_h[buf] = pltpu.async_copy(
                tok_v, out_hbm.at[pl.ds(b * S + s0 + q * QB, QB)], ssems[buf]
            )
        for buf in range(NBUF):
            if store_h[buf] is not None:
                store_h[buf].wait()

    out = emb(ids_flat, token_table, position_table)
    return out.reshape(B, S, D)


# speedup vs baseline: 1.1316x; 1.1316x over previous
import functools

import jax
import jax.numpy as jnp
from jax import lax
from jax.experimental import pallas as pl
from jax.experimental.pallas import tpu as pltpu
from jax.experimental.pallas import tpu_sc as plsc


def kernel(input_ids, token_table, position_table):
    B, S = input_ids.shape
    V, D = token_table.shape
    N = B * S
    L = 16

    info = plsc.get_sparse_core_info()
    NC, NS = info.num_cores, info.num_subcores
    NW = NC * NS
    SB = S // NW

    ids_flat = input_ids.reshape(N).astype(jnp.int32)
    mesh = plsc.VectorSubcoreMesh(core_axis_name="c", subcore_axis_name="s")

    @functools.partial(
        pl.kernel,
        mesh=mesh,
        out_type=jax.ShapeDtypeStruct((N, D), jnp.float32),
        scratch_types=[
            pltpu.VMEM((B * SB,), jnp.int32),
            pltpu.VMEM((SB, D), jnp.float32),
            pltpu.VMEM((SB, D), jnp.float32),
            pltpu.SemaphoreType.DMA,
        ],
    )
    def emb(ids_hbm, tok_hbm, pos_hbm, out_hbm, idx_v, pos_v, tok_v, sem):
        wid = lax.axis_index("s") * NC + lax.axis_index("c")
        s0 = wid * SB
        pltpu.sync_copy(pos_hbm.at[pl.ds(s0, SB)], pos_v)
        for b in range(B):
            pltpu.sync_copy(
                ids_hbm.at[pl.ds(b * S + s0, SB)], idx_v.at[pl.ds(b * SB, SB)]
            )
        for b in range(B):
            pltpu.async_copy(
                tok_hbm.at[idx_v.at[pl.ds(b * SB, SB)]], tok_v, sem
            ).wait()

            def row_add(r, carry):
                for j in range(D // L):
                    tok_v[r, pl.ds(j * L, L)] = (
                        tok_v[r, pl.ds(j * L, L)] + pos_v[r, pl.ds(j * L, L)]
                    )
                return carry

            lax.fori_loop(0, SB, row_add, 0)
            pltpu.sync_copy(tok_v, out_hbm.at[pl.ds(b * S + s0, SB)])

    out = emb(ids_flat, token_table, position_table)
    return out.reshape(B, S, D)
